# baseline (speedup 1.0000x reference)
import jax
import jax.numpy as jnp
from jax.experimental import pallas as pl
from jax.experimental.pallas import tpu as pltpu

_DIM = 1024
_SCALE = _DIM ** (-0.5)
_CH = 512
_NBUF = 6
_PRIME = _NBUF - 2


def _body(e_hbm, o_hbm, bufs, sin, sout):
    nch = e_hbm.shape[0] // _CH

    def in_copy(ch):
        b = ch % _NBUF
        return pltpu.make_async_copy(
            e_hbm.at[pl.ds(ch * _CH, _CH)], bufs.at[b], sin.at[b])

    def out_copy(ch):
        b = ch % _NBUF
        return pltpu.make_async_copy(
            bufs.at[b], o_hbm.at[pl.ds(ch * _CH, _CH)], sout.at[b])

    for ch in range(min(_PRIME, nch)):
        in_copy(ch).start()
    out_d = {}
    for ch in range(nch):
        b = ch % _NBUF
        nxt = ch + _PRIME
        if nxt < nch:
            if nxt >= _NBUF:
                out_d[nxt - _NBUF].wait()
            in_copy(nxt).start()
        in_copy(ch).wait()
        bufs[b] = bufs[b] * _SCALE
        d = out_copy(ch)
        d.start()
        out_d[ch] = d
    for ch in range(max(0, nch - _NBUF), nch):
        out_d[ch].wait()


def kernel(x, emb):
    seq_len = x.shape[1]
    emb = emb[:seq_len]
    return pl.pallas_call(
        _body,
        in_specs=[pl.BlockSpec(memory_space=pl.ANY)],
        out_specs=pl.BlockSpec(memory_space=pl.ANY),
        out_shape=jax.ShapeDtypeStruct((seq_len, _DIM), emb.dtype),
        scratch_shapes=[
            pltpu.VMEM((_NBUF, _CH, _DIM), jnp.float32),
            pltpu.SemaphoreType.DMA((_NBUF,)),
            pltpu.SemaphoreType.DMA((_NBUF,)),
        ],
    )(emb)
